# baseline (speedup 1.0000x reference)
import functools
import math

import jax
import jax.numpy as jnp
from jax import lax
from jax.experimental import pallas as pl
from jax.experimental.pallas import tpu as pltpu
from jax.experimental.pallas import tpu_sc as plsc

_R = 256
_C = 4096
_LOG2PI = math.log(2.0 * math.pi)
_BIG = 3.0e38


def _chol_params_col(cg, anc):
    isz = (jnp.max(jnp.abs(cg), axis=1, keepdims=True) == 0.0)

    def col(k):
        return cg[:, k:k + 1]

    mx = col(0) + anc[:, 0:1]
    my = col(1) + anc[:, 1:2]
    mz = col(2) + anc[:, 2:3]
    one = jnp.ones_like(col(3))
    zero = jnp.zeros_like(one)
    c11 = jnp.where(isz, one, col(3))
    c21 = jnp.where(isz, zero, col(6))
    c22 = jnp.where(isz, one, col(7))
    c31 = jnp.where(isz, zero, col(9))
    c32 = jnp.where(isz, zero, col(10))
    c33 = jnp.where(isz, one, col(11))
    l11 = jnp.sqrt(c11)
    l21 = c21 / l11
    l31 = c31 / l11
    l22 = jnp.sqrt(c22 - l21 * l21)
    l32 = (c32 - l31 * l21) / l22
    l33 = jnp.sqrt(c33 - l31 * l31 - l32 * l32)
    logdet = 2.0 * (jnp.log(l11) + jnp.log(l22) + jnp.log(l33))
    return (mx, my, mz), (l11, l21, l31, l22, l32, l33), logdet, isz


def _merge22(A, B):
    m0, M0 = jnp.minimum(A[0], B[0]), jnp.maximum(A[0], B[0])
    m1, M1 = jnp.minimum(A[1], B[1]), jnp.maximum(A[1], B[1])
    return [m0, jnp.minimum(M0, m1), jnp.maximum(M0, m1), M1]


def _merge44_5(A, B):
    d0 = jnp.minimum(A[0], B[0])
    x = jnp.maximum(A[0], B[0])
    y = jnp.minimum(A[2], B[2])
    d1 = jnp.minimum(x, y)
    d2 = jnp.maximum(x, y)
    e0 = jnp.minimum(A[1], B[1])
    u = jnp.maximum(A[1], B[1])
    v = jnp.minimum(A[3], B[3])
    e1 = jnp.minimum(u, v)
    return [d0, jnp.minimum(d1, e0), jnp.maximum(d1, e0),
            jnp.minimum(d2, e1), jnp.maximum(d2, e1)]


def _merge33_3(p, q):
    f0 = jnp.minimum(p[0], q[0])
    M0 = jnp.maximum(p[0], q[0])
    m1 = jnp.minimum(p[2], q[2])
    f1 = jnp.minimum(M0, m1)
    g0 = jnp.minimum(p[1], q[1])
    return [f0, jnp.minimum(f1, g0), jnp.maximum(f1, g0)]


def _merge55_5(a, b):
    d = _merge33_3([a[0], a[2], a[4]], [b[0], b[2], b[4]])
    e0 = jnp.minimum(a[1], b[1])
    M = jnp.maximum(a[1], b[1])
    m = jnp.minimum(a[3], b[3])
    e1 = jnp.minimum(M, m)
    return [d[0], jnp.minimum(d[1], e0), jnp.maximum(d[1], e0),
            jnp.minimum(d[2], e1), jnp.maximum(d[2], e1)]


def _stage1_body(nv, nc, vt_ref, o_ref, on_ref, at_ref, an_ref, cg_ref,
                 s_ref, w_ref, ai_ref, wm_ref, iz_ref):
    i = pl.program_id(0)
    vt = vt_ref[...]
    r = vt.shape[1]
    v2 = jnp.sum(vt * vt, axis=0, keepdims=True)
    cols = jax.lax.broadcasted_iota(jnp.int32, (1, r), 1)
    valid = (i * r + cols) < nv

    anc_t = at_ref[...]
    na = anc_t.shape[1]
    anc_n = an_ref[...]
    a2c = jnp.sum(anc_n * anc_n, axis=1, keepdims=True)
    va = jax.lax.dot_general(anc_t, vt, (((0,), (0,)), ((), ())),
                             preferred_element_type=jnp.float32,
                             precision=jax.lax.Precision.DEFAULT)
    ad2 = jnp.maximum(v2 + a2c - 2.0 * va, 0.0)
    amin = jnp.min(ad2, axis=0, keepdims=True)
    ia = jax.lax.broadcasted_iota(jnp.int32, (na, r), 0)
    aidx = jnp.min(jnp.where(ad2 == amin, ia, na), axis=0, keepdims=True)
    onehot = ia == aidx

    (mx, my, mz), (l11, l21, l31, l22, l32, l33), logdet, isz = \
        _chol_params_col(cg_ref[...], anc_n)
    dx = vt[0:1, :] - mx
    dy = vt[1:2, :] - my
    dz = vt[2:3, :] - mz
    y1 = dx / l11
    y2 = (dy - l21 * y1) / l22
    y3 = (dz - l31 * y1 - l32 * y2) / l33
    maha = y1 * y1 + y2 * y2 + y3 * y3
    logp = -0.5 * (3.0 * _LOG2PI + logdet + maha)
    wts = jnp.exp(logp)

    w_own = jnp.sum(jnp.where(onehot, wts, 0.0), axis=0, keepdims=True)
    wm_part = jnp.max(jnp.where(onehot & valid, wts, -_BIG), axis=1,
                      keepdims=True)

    @pl.when(i == 0)
    def _():
        wm_ref[...] = jnp.full_like(wm_ref, -_BIG)
    wm_ref[...] = jnp.maximum(wm_ref[...], wm_part[None])

    nv2 = -v2
    vt2 = vt + vt
    t = [jnp.full((1, r), _BIG, dtype=jnp.float32) for _ in range(5)]
    c = _C
    for ci in range(nc):
        o = o_ref[:, ci * c:(ci + 1) * c]
        on = on_ref[ci * c:(ci + 1) * c, :]
        o2c = jnp.sum(on * on, axis=1, keepdims=True)
        dot2 = jax.lax.dot_general(o, vt2, (((0,), (0,)), ((), ())),
                                   preferred_element_type=jnp.float32,
                                   precision=jax.lax.Precision.DEFAULT)
        key = o2c - dot2
        h = [key]
        n = c
        while n > 1:
            n //= 2
            A = [x[:n] for x in h]
            B = [x[n:] for x in h]
            if len(h) == 1:
                h = [jnp.minimum(A[0], B[0]), jnp.maximum(A[0], B[0])]
            elif len(h) == 2:
                h = _merge22(A, B)
            elif len(h) == 4:
                h = _merge44_5(A, B)
            else:
                h = _merge55_5(A, B)
        t = _merge55_5(t, h)

    s = 5.0 * v2 + sum(jnp.maximum(tk, nv2) for tk in t)
    s = jnp.where(valid, s, 0.0)

    s_ref[...] = s[None]
    w_ref[...] = w_own[None]
    ai_ref[...] = aidx[None]
    iz_ref[...] = isz.astype(jnp.float32).reshape(1, 1, na)


def _make_stage2_sc(nvp, nv):
    nw = 16
    per_w = nvp // nw
    nvec = per_w // 16
    mesh = plsc.VectorSubcoreMesh(core_axis_name="c", subcore_axis_name="s",
                                  num_cores=1)

    @functools.partial(
        pl.kernel, mesh=mesh,
        out_type=jax.ShapeDtypeStruct((16,), jnp.float32),
        scratch_types=[
            pltpu.VMEM((per_w,), jnp.float32),
            pltpu.VMEM((per_w,), jnp.float32),
            pltpu.VMEM((per_w,), jnp.int32),
            pltpu.VMEM((128,), jnp.float32),
            pltpu.VMEM_SHARED((nw * 16,), jnp.float32),
            pltpu.VMEM((nw * 16,), jnp.float32),
            pltpu.VMEM((16,), jnp.float32),
        ],
    )
    def stage2(s_hbm, w_hbm, ai_hbm, aux_hbm, out_hbm,
               s_v, w_v, ai_v, aux_v, shared, red_v, out_v):
        wid = lax.axis_index("s")
        base = wid * per_w
        pltpu.sync_copy(s_hbm.at[pl.ds(base, per_w)], s_v)
        pltpu.sync_copy(w_hbm.at[pl.ds(base, per_w)], w_v)
        pltpu.sync_copy(ai_hbm.at[pl.ds(base, per_w)], ai_v)
        pltpu.sync_copy(aux_hbm, aux_v)

        wt = aux_v[pl.ds(64, 16)]
        w0 = aux_v[pl.ds(0, 16)]
        w1 = aux_v[pl.ds(16, 16)]
        z0 = aux_v[pl.ds(32, 16)]
        z1 = aux_v[pl.ds(48, 16)]

        gdn = lax.GatherDimensionNumbers(
            offset_dims=(), collapsed_slice_dims=(0,), start_index_map=(0,))

        def take16(tbl, idx):
            return lax.gather(tbl, idx[:, None], gdn, (1,),
                              mode=lax.GatherScatterMode.PROMISE_IN_BOUNDS)

        def gather2(t0, t1, idx):
            lo = idx < 16
            i0 = jnp.where(lo, idx, 0)
            i1 = jnp.where(lo, 0, idx - 16)
            return jnp.where(lo, take16(t0, i0), take16(t1, i1))

        acc = jnp.zeros((16,), jnp.float32)
        for j in range(nvec):
            sl = pl.ds(j * 16, 16)
            sv = s_v[sl]
            wv = w_v[sl]
            av = ai_v[sl]
            wmv = gather2(w0, w1, av)
            izv = gather2(z0, z1, av)
            norm = jnp.where(wmv > 1.0, wv / wmv, wv)
            thr = jnp.where(norm > wt, norm, 0.0)
            acc = acc + jnp.where(izv > 0.5, 0.0, thr * thr * sv)

        out_v[...] = acc
        pltpu.sync_copy(out_v, shared.at[pl.ds(wid * 16, 16)])
        plsc.subcore_barrier()

        @pl.when(wid == 0)
        def _():
            pltpu.sync_copy(shared, red_v)
            acc2 = jnp.zeros((16,), jnp.float32)
            for k in range(nw):
                acc2 = acc2 + red_v[pl.ds(k * 16, 16)]
            lane = lax.broadcasted_iota(jnp.int32, (16,), 0)
            for sh in (8, 4, 2, 1):
                acc2 = acc2 + take16(acc2, lane ^ sh)
            out_v[...] = jnp.where(lane == 0, acc2 / (nv * 5.0), 0.0)
            pltpu.sync_copy(out_v, out_hbm)

    return stage2


def kernel(verts, anchor_verts, obj_pts, contact_gaussians, K, weights_threshold):
    nv = verts.shape[0]
    no = obj_pts.shape[0]
    na = anchor_verts.shape[0]
    nt = -(-nv // _R)
    nc = -(-no // _C)
    nvp = nt * _R
    nop = nc * _C

    v_t = jnp.zeros((4, nvp), jnp.float32).at[:3, :nv].set(verts.T)
    o_t = jnp.zeros((4, nop), jnp.float32)
    o_t = o_t.at[:3, :no].set(obj_pts.T).at[:3, no:].set(1.0e4)
    o_n = jnp.zeros((nop, 4), jnp.float32)
    o_n = o_n.at[:no, :3].set(obj_pts).at[no:, :3].set(1.0e4)
    a_t = jnp.zeros((4, na), jnp.float32).at[:3, :].set(anchor_verts.T)
    a_n = jnp.zeros((na, 4), jnp.float32).at[:, :3].set(anchor_verts)
    cg = contact_gaussians.astype(jnp.float32)

    s, w, ai, wm, iz = pl.pallas_call(
        functools.partial(_stage1_body, nv, nc),
        grid=(nt,),
        in_specs=[
            pl.BlockSpec((4, _R), lambda i: (0, i)),
            pl.BlockSpec((4, nop), lambda i: (0, 0)),
            pl.BlockSpec((nop, 4), lambda i: (0, 0)),
            pl.BlockSpec((4, na), lambda i: (0, 0)),
            pl.BlockSpec((na, 4), lambda i: (0, 0)),
            pl.BlockSpec((na, 12), lambda i: (0, 0)),
        ],
        out_specs=[
            pl.BlockSpec((1, 1, _R), lambda i: (i, 0, 0)),
            pl.BlockSpec((1, 1, _R), lambda i: (i, 0, 0)),
            pl.BlockSpec((1, 1, _R), lambda i: (i, 0, 0)),
            pl.BlockSpec((1, na, 1), lambda i: (0, 0, 0)),
            pl.BlockSpec((1, 1, na), lambda i: (0, 0, 0)),
        ],
        out_shape=[
            jax.ShapeDtypeStruct((nt, 1, _R), jnp.float32),
            jax.ShapeDtypeStruct((nt, 1, _R), jnp.float32),
            jax.ShapeDtypeStruct((nt, 1, _R), jnp.int32),
            jax.ShapeDtypeStruct((1, na, 1), jnp.float32),
            jax.ShapeDtypeStruct((1, 1, na), jnp.float32),
        ],
    )(v_t, o_t, o_n, a_t, a_n, cg)

    wt_vec = jnp.broadcast_to(
        jnp.asarray(weights_threshold, jnp.float32), (16,))
    aux = jnp.concatenate([wm.reshape(na), iz.reshape(na), wt_vec,
                           jnp.zeros((128 - 2 * na - 16,), jnp.float32)])
    out = _make_stage2_sc(nvp, float(nv))(
        s.reshape(nvp), w.reshape(nvp), ai.reshape(nvp), aux)
    return out[0]


# speedup vs baseline: 1.1358x; 1.1358x over previous
import functools
import math

import jax
import jax.numpy as jnp
from jax import lax
from jax.experimental import pallas as pl
from jax.experimental.pallas import tpu as pltpu
from jax.experimental.pallas import tpu_sc as plsc

_R = 512
_C = 4096
_LOG2PI = math.log(2.0 * math.pi)
_BIG = 3.0e38


def _chol_params_col(cg, anc):
    isz = (jnp.max(jnp.abs(cg), axis=1, keepdims=True) == 0.0)

    def col(k):
        return cg[:, k:k + 1]

    mx = col(0) + anc[:, 0:1]
    my = col(1) + anc[:, 1:2]
    mz = col(2) + anc[:, 2:3]
    one = jnp.ones_like(col(3))
    zero = jnp.zeros_like(one)
    c11 = jnp.where(isz, one, col(3))
    c21 = jnp.where(isz, zero, col(6))
    c22 = jnp.where(isz, one, col(7))
    c31 = jnp.where(isz, zero, col(9))
    c32 = jnp.where(isz, zero, col(10))
    c33 = jnp.where(isz, one, col(11))
    l11 = jnp.sqrt(c11)
    l21 = c21 / l11
    l31 = c31 / l11
    l22 = jnp.sqrt(c22 - l21 * l21)
    l32 = (c32 - l31 * l21) / l22
    l33 = jnp.sqrt(c33 - l31 * l31 - l32 * l32)
    logdet = 2.0 * (jnp.log(l11) + jnp.log(l22) + jnp.log(l33))
    return (mx, my, mz), (l11, l21, l31, l22, l32, l33), logdet, isz


def _merge22(A, B):
    m0, M0 = jnp.minimum(A[0], B[0]), jnp.maximum(A[0], B[0])
    m1, M1 = jnp.minimum(A[1], B[1]), jnp.maximum(A[1], B[1])
    return [m0, jnp.minimum(M0, m1), jnp.maximum(M0, m1), M1]


def _merge44_5(A, B):
    d0 = jnp.minimum(A[0], B[0])
    x = jnp.maximum(A[0], B[0])
    y = jnp.minimum(A[2], B[2])
    d1 = jnp.minimum(x, y)
    d2 = jnp.maximum(x, y)
    e0 = jnp.minimum(A[1], B[1])
    u = jnp.maximum(A[1], B[1])
    v = jnp.minimum(A[3], B[3])
    e1 = jnp.minimum(u, v)
    return [d0, jnp.minimum(d1, e0), jnp.maximum(d1, e0),
            jnp.minimum(d2, e1), jnp.maximum(d2, e1)]


def _merge33_3(p, q):
    f0 = jnp.minimum(p[0], q[0])
    M0 = jnp.maximum(p[0], q[0])
    m1 = jnp.minimum(p[2], q[2])
    f1 = jnp.minimum(M0, m1)
    g0 = jnp.minimum(p[1], q[1])
    return [f0, jnp.minimum(f1, g0), jnp.maximum(f1, g0)]


def _merge55_5(a, b):
    d = _merge33_3([a[0], a[2], a[4]], [b[0], b[2], b[4]])
    e0 = jnp.minimum(a[1], b[1])
    M = jnp.maximum(a[1], b[1])
    m = jnp.minimum(a[3], b[3])
    e1 = jnp.minimum(M, m)
    return [d[0], jnp.minimum(d[1], e0), jnp.maximum(d[1], e0),
            jnp.minimum(d[2], e1), jnp.maximum(d[2], e1)]


def _stage1_body(nv, nc, vt_ref, o_ref, on_ref, at_ref, an_ref, cg_ref,
                 s_ref, w_ref, ai_ref, wm_ref, iz_ref):
    i = pl.program_id(0)
    vt = vt_ref[...]
    r = vt.shape[1]
    v2 = jnp.sum(vt * vt, axis=0, keepdims=True)
    cols = jax.lax.broadcasted_iota(jnp.int32, (1, r), 1)
    valid = (i * r + cols) < nv

    anc_t = at_ref[...]
    na = anc_t.shape[1]
    anc_n = an_ref[...]
    a2c = jnp.sum(anc_n * anc_n, axis=1, keepdims=True)
    va = jax.lax.dot_general(anc_t, vt, (((0,), (0,)), ((), ())),
                             preferred_element_type=jnp.float32,
                             precision=jax.lax.Precision.DEFAULT)
    ad2 = jnp.maximum(v2 + a2c - 2.0 * va, 0.0)
    amin = jnp.min(ad2, axis=0, keepdims=True)
    ia = jax.lax.broadcasted_iota(jnp.int32, (na, r), 0)
    aidx = jnp.min(jnp.where(ad2 == amin, ia, na), axis=0, keepdims=True)
    onehot = ia == aidx

    (mx, my, mz), (l11, l21, l31, l22, l32, l33), logdet, isz = \
        _chol_params_col(cg_ref[...], anc_n)
    dx = vt[0:1, :] - mx
    dy = vt[1:2, :] - my
    dz = vt[2:3, :] - mz
    y1 = dx / l11
    y2 = (dy - l21 * y1) / l22
    y3 = (dz - l31 * y1 - l32 * y2) / l33
    maha = y1 * y1 + y2 * y2 + y3 * y3
    logp = -0.5 * (3.0 * _LOG2PI + logdet + maha)
    wts = jnp.exp(logp)

    w_own = jnp.sum(jnp.where(onehot, wts, 0.0), axis=0, keepdims=True)
    wm_part = jnp.max(jnp.where(onehot & valid, wts, -_BIG), axis=1,
                      keepdims=True)

    @pl.when(i == 0)
    def _():
        wm_ref[...] = jnp.full_like(wm_ref, -_BIG)
    wm_ref[...] = jnp.maximum(wm_ref[...], wm_part[None])

    nv2 = -v2
    vt2 = vt + vt
    t = [jnp.full((1, r), _BIG, dtype=jnp.float32) for _ in range(5)]
    c = _C
    for ci in range(nc):
        o = o_ref[:, ci * c:(ci + 1) * c]
        on = on_ref[ci * c:(ci + 1) * c, :]
        o2c = jnp.sum(on * on, axis=1, keepdims=True)
        dot2 = jax.lax.dot_general(o, vt2, (((0,), (0,)), ((), ())),
                                   preferred_element_type=jnp.float32,
                                   precision=jax.lax.Precision.DEFAULT)
        key = o2c - dot2
        h = [key]
        n = c
        while n > 1:
            n //= 2
            A = [x[:n] for x in h]
            B = [x[n:] for x in h]
            if len(h) == 1:
                h = [jnp.minimum(A[0], B[0]), jnp.maximum(A[0], B[0])]
            elif len(h) == 2:
                h = _merge22(A, B)
            elif len(h) == 4:
                h = _merge44_5(A, B)
            else:
                h = _merge55_5(A, B)
        t = _merge55_5(t, h)

    s = 5.0 * v2 + sum(jnp.maximum(tk, nv2) for tk in t)
    s = jnp.where(valid, s, 0.0)

    s_ref[...] = s[None]
    w_ref[...] = w_own[None]
    ai_ref[...] = aidx[None]
    iz_ref[...] = isz.astype(jnp.float32).reshape(1, 1, na)


def _make_stage2_sc(nvp, nv):
    nw = 16
    per_w = nvp // nw
    nvec = per_w // 16
    mesh = plsc.VectorSubcoreMesh(core_axis_name="c", subcore_axis_name="s",
                                  num_cores=1)

    @functools.partial(
        pl.kernel, mesh=mesh,
        out_type=jax.ShapeDtypeStruct((16,), jnp.float32),
        scratch_types=[
            pltpu.VMEM((per_w,), jnp.float32),
            pltpu.VMEM((per_w,), jnp.float32),
            pltpu.VMEM((per_w,), jnp.int32),
            pltpu.VMEM((128,), jnp.float32),
            pltpu.VMEM_SHARED((nw * 16,), jnp.float32),
            pltpu.VMEM((nw * 16,), jnp.float32),
            pltpu.VMEM((16,), jnp.float32),
        ],
    )
    def stage2(s_hbm, w_hbm, ai_hbm, aux_hbm, out_hbm,
               s_v, w_v, ai_v, aux_v, shared, red_v, out_v):
        wid = lax.axis_index("s")
        base = wid * per_w
        pltpu.sync_copy(s_hbm.at[pl.ds(base, per_w)], s_v)
        pltpu.sync_copy(w_hbm.at[pl.ds(base, per_w)], w_v)
        pltpu.sync_copy(ai_hbm.at[pl.ds(base, per_w)], ai_v)
        pltpu.sync_copy(aux_hbm, aux_v)

        wt = aux_v[pl.ds(64, 16)]
        w0 = aux_v[pl.ds(0, 16)]
        w1 = aux_v[pl.ds(16, 16)]
        z0 = aux_v[pl.ds(32, 16)]
        z1 = aux_v[pl.ds(48, 16)]

        gdn = lax.GatherDimensionNumbers(
            offset_dims=(), collapsed_slice_dims=(0,), start_index_map=(0,))

        def take16(tbl, idx):
            return lax.gather(tbl, idx[:, None], gdn, (1,),
                              mode=lax.GatherScatterMode.PROMISE_IN_BOUNDS)

        def gather2(t0, t1, idx):
            lo = idx < 16
            i0 = jnp.where(lo, idx, 0)
            i1 = jnp.where(lo, 0, idx - 16)
            return jnp.where(lo, take16(t0, i0), take16(t1, i1))

        acc = jnp.zeros((16,), jnp.float32)
        for j in range(nvec):
            sl = pl.ds(j * 16, 16)
            sv = s_v[sl]
            wv = w_v[sl]
            av = ai_v[sl]
            wmv = gather2(w0, w1, av)
            izv = gather2(z0, z1, av)
            norm = jnp.where(wmv > 1.0, wv / wmv, wv)
            thr = jnp.where(norm > wt, norm, 0.0)
            acc = acc + jnp.where(izv > 0.5, 0.0, thr * thr * sv)

        out_v[...] = acc
        pltpu.sync_copy(out_v, shared.at[pl.ds(wid * 16, 16)])
        plsc.subcore_barrier()

        @pl.when(wid == 0)
        def _():
            pltpu.sync_copy(shared, red_v)
            acc2 = jnp.zeros((16,), jnp.float32)
            for k in range(nw):
                acc2 = acc2 + red_v[pl.ds(k * 16, 16)]
            lane = lax.broadcasted_iota(jnp.int32, (16,), 0)
            for sh in (8, 4, 2, 1):
                acc2 = acc2 + take16(acc2, lane ^ sh)
            out_v[...] = jnp.where(lane == 0, acc2 / (nv * 5.0), 0.0)
            pltpu.sync_copy(out_v, out_hbm)

    return stage2


def kernel(verts, anchor_verts, obj_pts, contact_gaussians, K, weights_threshold):
    nv = verts.shape[0]
    no = obj_pts.shape[0]
    na = anchor_verts.shape[0]
    nt = -(-nv // _R)
    nc = -(-no // _C)
    nvp = nt * _R
    nop = nc * _C

    v_t = jnp.zeros((4, nvp), jnp.float32).at[:3, :nv].set(verts.T)
    o_t = jnp.zeros((4, nop), jnp.float32)
    o_t = o_t.at[:3, :no].set(obj_pts.T).at[:3, no:].set(1.0e4)
    o_n = jnp.zeros((nop, 4), jnp.float32)
    o_n = o_n.at[:no, :3].set(obj_pts).at[no:, :3].set(1.0e4)
    a_t = jnp.zeros((4, na), jnp.float32).at[:3, :].set(anchor_verts.T)
    a_n = jnp.zeros((na, 4), jnp.float32).at[:, :3].set(anchor_verts)
    cg = contact_gaussians.astype(jnp.float32)

    s, w, ai, wm, iz = pl.pallas_call(
        functools.partial(_stage1_body, nv, nc),
        grid=(nt,),
        in_specs=[
            pl.BlockSpec((4, _R), lambda i: (0, i)),
            pl.BlockSpec((4, nop), lambda i: (0, 0)),
            pl.BlockSpec((nop, 4), lambda i: (0, 0)),
            pl.BlockSpec((4, na), lambda i: (0, 0)),
            pl.BlockSpec((na, 4), lambda i: (0, 0)),
            pl.BlockSpec((na, 12), lambda i: (0, 0)),
        ],
        out_specs=[
            pl.BlockSpec((1, 1, _R), lambda i: (i, 0, 0)),
            pl.BlockSpec((1, 1, _R), lambda i: (i, 0, 0)),
            pl.BlockSpec((1, 1, _R), lambda i: (i, 0, 0)),
            pl.BlockSpec((1, na, 1), lambda i: (0, 0, 0)),
            pl.BlockSpec((1, 1, na), lambda i: (0, 0, 0)),
        ],
        out_shape=[
            jax.ShapeDtypeStruct((nt, 1, _R), jnp.float32),
            jax.ShapeDtypeStruct((nt, 1, _R), jnp.float32),
            jax.ShapeDtypeStruct((nt, 1, _R), jnp.int32),
            jax.ShapeDtypeStruct((1, na, 1), jnp.float32),
            jax.ShapeDtypeStruct((1, 1, na), jnp.float32),
        ],
    )(v_t, o_t, o_n, a_t, a_n, cg)

    wt_vec = jnp.broadcast_to(
        jnp.asarray(weights_threshold, jnp.float32), (16,))
    aux = jnp.concatenate([wm.reshape(na), iz.reshape(na), wt_vec,
                           jnp.zeros((128 - 2 * na - 16,), jnp.float32)])
    out = _make_stage2_sc(nvp, float(nv))(
        s.reshape(nvp), w.reshape(nvp), ai.reshape(nvp), aux)
    return out[0]
